# baseline (speedup 1.0000x reference)
import functools

import jax
import jax.numpy as jnp
from jax import lax
from jax.experimental import pallas as pl
from jax.experimental.pallas import tpu as pltpu
from jax.experimental.pallas import tpu_sc as plsc

ATOMS = 51
D = 128
N = 4096 * 200
NC, NS, L = 2, 16, 16
NW = NC * NS
B_LOC = N // NW
BLK = 256
NBLK = B_LOC // BLK
TBL_WORDS = ATOMS * D


def _make_kernel():
    mesh = plsc.VectorSubcoreMesh(core_axis_name="c", subcore_axis_name="s")

    @functools.partial(
        pl.kernel,
        mesh=mesh,
        out_type=jax.ShapeDtypeStruct((N * D,), jnp.float32),
        compiler_params=pltpu.CompilerParams(needs_layout_passes=False),
        scratch_types=[
            pltpu.VMEM((B_LOC,), jnp.float32),
            pltpu.VMEM((TBL_WORDS,), jnp.float32),
            pltpu.VMEM((BLK * D,), jnp.float32),
        ],
    )
    def body(x_hbm, tbl_hbm, out_hbm, x_v, tbl_v, ob):
        wid = lax.axis_index("s") * NC + lax.axis_index("c")
        base = wid * B_LOC
        pltpu.sync_copy(tbl_hbm, tbl_v)
        pltpu.sync_copy(x_hbm.at[pl.ds(base, B_LOC)], x_v)

        lane = lax.iota(jnp.int32, L)

        def block_body(b, _):
            def group_body(g, _):
                e0 = b * BLK + g * L
                xv = x_v[pl.ds(e0, L)]
                xs = jnp.clip(xv * jnp.float32(ATOMS), 0.0, float(ATOMS - 1))
                fl = xs.astype(jnp.int32)
                d = xs - fl.astype(jnp.float32)
                ce = jnp.minimum(fl + 1, ATOMS - 1)
                ia = fl * D
                ic = ce * D
                io = (g * L + lane) * D

                for f in range(D):
                    a = plsc.load_gather(tbl_v, [ia + f])
                    c = plsc.load_gather(tbl_v, [ic + f])
                    r = a + d * (c - a)
                    plsc.store_scatter(ob, [io + f], r)
                return _

            lax.fori_loop(0, BLK // L, group_body, 0)
            pltpu.sync_copy(ob, out_hbm.at[pl.ds((base + b * BLK) * D, BLK * D)])
            return _

        lax.fori_loop(0, NBLK, block_body, 0)

    return body


_kernel_fn = _make_kernel()


def kernel(x, table):
    xf = x.reshape(N)
    tf = table.reshape(TBL_WORDS)
    out = _kernel_fn(xf, tf)
    return out.reshape(x.shape[0], x.shape[1], D)


# speedup vs baseline: 5.0889x; 5.0889x over previous
import functools

import jax
import jax.numpy as jnp
from jax import lax
from jax.experimental import pallas as pl
from jax.experimental.pallas import tpu as pltpu
from jax.experimental.pallas import tpu_sc as plsc

ATOMS = 51
D = 128
N = 4096 * 200
NC, NS, L = 2, 16, 16
NW = NC * NS
B_LOC = N // NW
BLK = 256
NBLK = B_LOC // BLK
ROW = 2 * D
TAD_WORDS = ATOMS * ROW


def _make_kernel():
    mesh = plsc.VectorSubcoreMesh(core_axis_name="c", subcore_axis_name="s")

    @functools.partial(
        pl.kernel,
        mesh=mesh,
        out_type=jax.ShapeDtypeStruct((N * D,), jnp.float32),
        compiler_params=pltpu.CompilerParams(needs_layout_passes=False),
        scratch_types=[
            pltpu.VMEM((B_LOC,), jnp.float32),
            pltpu.VMEM((TAD_WORDS,), jnp.float32),
            pltpu.VMEM((BLK * D,), jnp.float32),
            pltpu.VMEM((BLK * D,), jnp.float32),
            pltpu.SemaphoreType.DMA,
            pltpu.SemaphoreType.DMA,
        ],
    )
    def body(x_hbm, tad_hbm, out_hbm, x_v, tad_v, ob0, ob1, sem0, sem1):
        wid = lax.axis_index("s") * NC + lax.axis_index("c")
        base = wid * B_LOC
        pltpu.sync_copy(tad_hbm, tad_v)
        pltpu.sync_copy(x_hbm.at[pl.ds(base, B_LOC)], x_v)

        def compute_block(b, ob):
            def group_body(g, _):
                e0 = b * BLK + g * L
                xv = x_v[pl.ds(e0, L)]
                xs = jnp.clip(xv * jnp.float32(ATOMS), 0.0, float(ATOMS - 1))
                fl = xs.astype(jnp.int32)
                d = xs - fl.astype(jnp.float32)
                bv = fl * ROW
                for e in range(L):
                    be = bv[e]
                    de = lax.broadcast(d[e], (L,))
                    o0 = (g * L + e) * D
                    for c in range(D // L):
                        av = tad_v[pl.ds(be + c * L, L)]
                        dv = tad_v[pl.ds(be + (D + c * L), L)]
                        ob[pl.ds(o0 + c * L, L)] = av + de * dv
                return _

            lax.fori_loop(0, BLK // L, group_body, 0, unroll=False)

        def out_slice(b):
            return out_hbm.at[pl.ds((base + b * BLK) * D, BLK * D)]

        def wait(ob, sem):
            pltpu.make_async_copy(ob, out_slice(0), sem).wait()

        def pair_body(i, _):
            b0 = 2 * i

            @pl.when(i > 0)
            def _w0():
                wait(ob0, sem0)

            compute_block(b0, ob0)
            pltpu.async_copy(ob0, out_slice(b0), sem0)

            @pl.when(i > 0)
            def _w1():
                wait(ob1, sem1)

            compute_block(b0 + 1, ob1)
            pltpu.async_copy(ob1, out_slice(b0 + 1), sem1)
            return _

        lax.fori_loop(0, NBLK // 2, pair_body, 0, unroll=False)
        wait(ob0, sem0)
        wait(ob1, sem1)

    return body


_kernel_fn = _make_kernel()


def kernel(x, table):
    xf = x.reshape(N)
    tpad = jnp.concatenate([table, table[-1:]], axis=0)
    tad = jnp.concatenate([tpad[:-1], tpad[1:] - tpad[:-1]], axis=1)
    out = _kernel_fn(xf, tad.reshape(TAD_WORDS))
    return out.reshape(x.shape[0], x.shape[1], D)


# speedup vs baseline: 20.3212x; 3.9933x over previous
import functools

import jax
import jax.numpy as jnp
from jax import lax
from jax.experimental import pallas as pl
from jax.experimental.pallas import tpu as pltpu
from jax.experimental.pallas import tpu_sc as plsc

ATOMS = 51
D = 128
N = 4096 * 200
NC, NS, L = 2, 16, 16
NW = NC * NS
B_LOC = N // NW
BLK = 256
NBLK = B_LOC // BLK
ROW = 2 * D
TAD_WORDS = ATOMS * ROW


def _make_kernel():
    mesh = plsc.VectorSubcoreMesh(core_axis_name="c", subcore_axis_name="s")

    @functools.partial(
        pl.kernel,
        mesh=mesh,
        out_type=jax.ShapeDtypeStruct((N * D,), jnp.float32),
        compiler_params=pltpu.CompilerParams(needs_layout_passes=False),
        scratch_types=[
            pltpu.VMEM((B_LOC,), jnp.float32),
            pltpu.VMEM((TAD_WORDS,), jnp.float32),
            pltpu.VMEM((BLK,), jnp.int32),
            pltpu.VMEM((BLK,), jnp.float32),
            pltpu.VMEM((BLK * D,), jnp.float32),
            pltpu.VMEM((BLK * D,), jnp.float32),
            pltpu.SemaphoreType.DMA,
            pltpu.SemaphoreType.DMA,
        ],
    )
    def body(x_hbm, tad_hbm, out_hbm, x_v, tad_v, bvb, dvb, ob0, ob1, sem0, sem1):
        wid = lax.axis_index("s") * NC + lax.axis_index("c")
        base = wid * B_LOC
        pltpu.sync_copy(tad_hbm, tad_v)
        pltpu.sync_copy(x_hbm.at[pl.ds(base, B_LOC)], x_v)

        lane = lax.iota(jnp.int32, L)
        ka = [lane + c * L for c in range(D // L)]
        kd = [lane + (D + c * L) for c in range(D // L)]

        def compute_block(b, ob):
            def group_body(g, _):
                xv = x_v[pl.ds(b * BLK + g * L, L)]
                xs = jnp.clip(xv * jnp.float32(ATOMS), 0.0, float(ATOMS - 1))
                fl = xs.astype(jnp.int32)
                d = xs - fl.astype(jnp.float32)
                bvb[pl.ds(g * L, L)] = fl * ROW
                dvb[pl.ds(g * L, L)] = d
                return _

            lax.fori_loop(0, BLK // L, group_body, 0, unroll=True)

            @plsc.parallel_loop(0, BLK, unroll=4)
            def elem_body(e):
                es = lax.broadcast(e, (L,))
                bsp = plsc.load_gather(bvb, [es])
                dsp = plsc.load_gather(dvb, [es])
                o0 = e * D
                for c in range(D // L):
                    av = plsc.load_gather(tad_v, [bsp + ka[c]])
                    dv = plsc.load_gather(tad_v, [bsp + kd[c]])
                    ob[pl.ds(o0 + c * L, L)] = av + dsp * dv

        def out_slice(b):
            return out_hbm.at[pl.ds((base + b * BLK) * D, BLK * D)]

        def wait(ob, sem):
            pltpu.make_async_copy(ob, out_slice(0), sem).wait()

        def pair_body(i, _):
            b0 = 2 * i

            @pl.when(i > 0)
            def _w0():
                wait(ob0, sem0)

            compute_block(b0, ob0)
            pltpu.async_copy(ob0, out_slice(b0), sem0)

            @pl.when(i > 0)
            def _w1():
                wait(ob1, sem1)

            compute_block(b0 + 1, ob1)
            pltpu.async_copy(ob1, out_slice(b0 + 1), sem1)
            return _

        lax.fori_loop(0, NBLK // 2, pair_body, 0, unroll=False)
        wait(ob0, sem0)
        wait(ob1, sem1)

    return body


_kernel_fn = _make_kernel()


def kernel(x, table):
    xf = x.reshape(N)
    tpad = jnp.concatenate([table, table[-1:]], axis=0)
    tad = jnp.concatenate([tpad[:-1], tpad[1:] - tpad[:-1]], axis=1)
    out = _kernel_fn(xf, tad.reshape(TAD_WORDS))
    return out.reshape(x.shape[0], x.shape[1], D)


# speedup vs baseline: 24.6625x; 1.2136x over previous
import functools

import jax
import jax.numpy as jnp
from jax import lax
from jax.experimental import pallas as pl
from jax.experimental.pallas import tpu as pltpu
from jax.experimental.pallas import tpu_sc as plsc

ATOMS = 51
D = 128
N = 4096 * 200
NC, NS, L = 2, 16, 16
NW = NC * NS
B_LOC = N // NW
BLK = 256
NBLK = B_LOC // BLK
TAD_WORDS = ATOMS * D


def _make_kernel():
    mesh = plsc.VectorSubcoreMesh(core_axis_name="c", subcore_axis_name="s")

    @functools.partial(
        pl.kernel,
        mesh=mesh,
        out_type=jax.ShapeDtypeStruct((N * D,), jnp.float32),
        compiler_params=pltpu.CompilerParams(needs_layout_passes=False),
        scratch_types=[
            pltpu.VMEM((B_LOC,), jnp.float32),
            pltpu.VMEM((TAD_WORDS,), jnp.int32),
            pltpu.VMEM((BLK,), jnp.int32),
            pltpu.VMEM((BLK,), jnp.float32),
            pltpu.VMEM((BLK * D,), jnp.float32),
            pltpu.VMEM((BLK * D,), jnp.float32),
            pltpu.SemaphoreType.DMA,
            pltpu.SemaphoreType.DMA,
        ],
    )
    def body(x_hbm, tad_hbm, out_hbm, x_v, tad_v, bvb, dvb, ob0, ob1, sem0, sem1):
        wid = lax.axis_index("s") * NC + lax.axis_index("c")
        base = wid * B_LOC
        pltpu.sync_copy(tad_hbm, tad_v)
        pltpu.sync_copy(x_hbm.at[pl.ds(base, B_LOC)], x_v)

        lane = lax.iota(jnp.int32, L)
        ka = [lane + c * L for c in range(D // L)]
        himask = jnp.full((L,), -65536, jnp.int32)

        def compute_block(b, ob):
            def group_body(g, _):
                xv = x_v[pl.ds(b * BLK + g * L, L)]
                xs = jnp.clip(xv * jnp.float32(ATOMS), 0.0, float(ATOMS - 1))
                fl = xs.astype(jnp.int32)
                d = xs - fl.astype(jnp.float32)
                bvb[pl.ds(g * L, L)] = fl * D
                dvb[pl.ds(g * L, L)] = d
                return _

            lax.fori_loop(0, BLK // L, group_body, 0, unroll=True)

            @plsc.parallel_loop(0, BLK, unroll=4)
            def elem_body(e):
                es = lax.broadcast(e, (L,))
                bsp = plsc.load_gather(bvb, [es])
                dsp = plsc.load_gather(dvb, [es])
                o0 = e * D
                for c in range(D // L):
                    w = plsc.load_gather(tad_v, [bsp + ka[c]])
                    av = lax.bitcast_convert_type(w & himask, jnp.float32)
                    dv = lax.bitcast_convert_type(
                        lax.shift_left(w, 16), jnp.float32)
                    ob[pl.ds(o0 + c * L, L)] = av + dsp * dv

        def out_slice(b):
            return out_hbm.at[pl.ds((base + b * BLK) * D, BLK * D)]

        def wait(ob, sem):
            pltpu.make_async_copy(ob, out_slice(0), sem).wait()

        def pair_body(i, _):
            b0 = 2 * i

            @pl.when(i > 0)
            def _w0():
                wait(ob0, sem0)

            compute_block(b0, ob0)
            pltpu.async_copy(ob0, out_slice(b0), sem0)

            @pl.when(i > 0)
            def _w1():
                wait(ob1, sem1)

            compute_block(b0 + 1, ob1)
            pltpu.async_copy(ob1, out_slice(b0 + 1), sem1)
            return _

        lax.fori_loop(0, NBLK // 2, pair_body, 0, unroll=False)
        wait(ob0, sem0)
        wait(ob1, sem1)

    return body


_kernel_fn = _make_kernel()


def kernel(x, table):
    xf = x.reshape(N)
    tpad = jnp.concatenate([table, table[-1:]], axis=0)
    a_bits = jax.lax.bitcast_convert_type(
        tpad[:-1].astype(jnp.bfloat16), jnp.uint16).astype(jnp.uint32)
    d_bits = jax.lax.bitcast_convert_type(
        (tpad[1:] - tpad[:-1]).astype(jnp.bfloat16), jnp.uint16
    ).astype(jnp.uint32)
    tad = jax.lax.bitcast_convert_type((a_bits << 16) | d_bits, jnp.int32)
    out = _kernel_fn(xf, tad.reshape(TAD_WORDS))
    return out.reshape(x.shape[0], x.shape[1], D)


# speedup vs baseline: 27.8595x; 1.1296x over previous
import functools

import jax
import jax.numpy as jnp
from jax import lax
from jax.experimental import pallas as pl
from jax.experimental.pallas import tpu as pltpu
from jax.experimental.pallas import tpu_sc as plsc

ATOMS = 51
D = 128
N = 4096 * 200
NC, NS, L = 2, 16, 16
NW = NC * NS
B_LOC = N // NW
BLK = 256
NBLK = B_LOC // BLK
TAD_WORDS = ATOMS * D


def _make_kernel():
    mesh = plsc.VectorSubcoreMesh(core_axis_name="c", subcore_axis_name="s")

    @functools.partial(
        pl.kernel,
        mesh=mesh,
        out_type=jax.ShapeDtypeStruct((N * D,), jnp.float32),
        compiler_params=pltpu.CompilerParams(needs_layout_passes=False),
        scratch_types=[
            pltpu.VMEM((B_LOC,), jnp.float32),
            pltpu.VMEM((TAD_WORDS,), jnp.int32),
            pltpu.VMEM((BLK,), jnp.int32),
            pltpu.VMEM((BLK,), jnp.float32),
            pltpu.VMEM((BLK * D,), jnp.float32),
            pltpu.VMEM((BLK * D,), jnp.float32),
            pltpu.SemaphoreType.DMA,
            pltpu.SemaphoreType.DMA,
        ],
    )
    def body(x_hbm, tad_hbm, out_hbm, x_v, tad_v, bvb, dvb, ob0, ob1, sem0, sem1):
        wid = lax.axis_index("s") * NC + lax.axis_index("c")
        base = wid * B_LOC
        pltpu.sync_copy(tad_hbm, tad_v)
        pltpu.sync_copy(x_hbm.at[pl.ds(base, B_LOC)], x_v)

        lane = lax.iota(jnp.int32, L)
        ka = [lane + c * L for c in range(D // L)]

        def compute_block(b, ob):
            def group_body(g, _):
                xv = x_v[pl.ds(b * BLK + g * L, L)]
                xs = jnp.clip(xv * jnp.float32(ATOMS), 0.0, float(ATOMS - 1))
                fl = xs.astype(jnp.int32)
                d = xs - fl.astype(jnp.float32)
                bvb[pl.ds(g * L, L)] = fl * D
                dvb[pl.ds(g * L, L)] = d
                return _

            lax.fori_loop(0, BLK // L, group_body, 0, unroll=True)

            @plsc.parallel_loop(0, BLK, unroll=8)
            def elem_body(e):
                es = lax.broadcast(e, (L,))
                bsp = plsc.load_gather(bvb, [es])
                dsp = plsc.load_gather(dvb, [es])
                o0 = e * D
                for c in range(D // L):
                    w = plsc.load_gather(tad_v, [bsp + ka[c]])
                    av = lax.bitcast_convert_type(w, jnp.float32)
                    dv = lax.bitcast_convert_type(
                        lax.shift_left(w, 16), jnp.float32)
                    ob[pl.ds(o0 + c * L, L)] = av + dsp * dv

        def out_slice(b):
            return out_hbm.at[pl.ds((base + b * BLK) * D, BLK * D)]

        def wait(ob, sem):
            pltpu.make_async_copy(ob, out_slice(0), sem).wait()

        def pair_body(i, _):
            b0 = 2 * i

            @pl.when(i > 0)
            def _w0():
                wait(ob0, sem0)

            compute_block(b0, ob0)
            pltpu.async_copy(ob0, out_slice(b0), sem0)

            @pl.when(i > 0)
            def _w1():
                wait(ob1, sem1)

            compute_block(b0 + 1, ob1)
            pltpu.async_copy(ob1, out_slice(b0 + 1), sem1)
            return _

        lax.fori_loop(0, NBLK // 2, pair_body, 0, unroll=False)
        wait(ob0, sem0)
        wait(ob1, sem1)

    return body


_kernel_fn = _make_kernel()


def kernel(x, table):
    xf = x.reshape(N)
    tpad = jnp.concatenate([table, table[-1:]], axis=0)
    a_bits = jax.lax.bitcast_convert_type(
        tpad[:-1].astype(jnp.bfloat16), jnp.uint16).astype(jnp.uint32)
    d_bits = jax.lax.bitcast_convert_type(
        (tpad[1:] - tpad[:-1]).astype(jnp.bfloat16), jnp.uint16
    ).astype(jnp.uint32)
    tad = jax.lax.bitcast_convert_type((a_bits << 16) | d_bits, jnp.int32)
    out = _kernel_fn(xf, tad.reshape(TAD_WORDS))
    return out.reshape(x.shape[0], x.shape[1], D)


# speedup vs baseline: 27.8969x; 1.0013x over previous
import functools

import jax
import jax.numpy as jnp
from jax import lax
from jax.experimental import pallas as pl
from jax.experimental.pallas import tpu as pltpu
from jax.experimental.pallas import tpu_sc as plsc

ATOMS = 51
D = 128
N = 4096 * 200
NC, NS, L = 2, 16, 16
NW = NC * NS
B_LOC = N // NW
BLK = 256
NBLK = B_LOC // BLK
TAD_WORDS = ATOMS * D


def _make_kernel():
    mesh = plsc.VectorSubcoreMesh(core_axis_name="c", subcore_axis_name="s")

    @functools.partial(
        pl.kernel,
        mesh=mesh,
        out_type=jax.ShapeDtypeStruct((N * D,), jnp.float32),
        compiler_params=pltpu.CompilerParams(needs_layout_passes=False),
        scratch_types=[
            pltpu.VMEM((B_LOC,), jnp.float32),
            pltpu.VMEM((TAD_WORDS,), jnp.int32),
            pltpu.VMEM((BLK,), jnp.int32),
            pltpu.VMEM((BLK,), jnp.float32),
            pltpu.VMEM((BLK * D,), jnp.float32),
            pltpu.VMEM((BLK * D,), jnp.float32),
            pltpu.SemaphoreType.DMA,
            pltpu.SemaphoreType.DMA,
        ],
    )
    def body(x_hbm, tad_hbm, out_hbm, x_v, tad_v, bvb, dvb, ob0, ob1, sem0, sem1):
        wid = lax.axis_index("s") * NC + lax.axis_index("c")
        base = wid * B_LOC
        pltpu.sync_copy(tad_hbm, tad_v)
        pltpu.sync_copy(x_hbm.at[pl.ds(base, B_LOC)], x_v)

        lane = lax.iota(jnp.int32, L)
        ka = [lane + c * L for c in range(D // L)]
        himask = jnp.full((L,), -65536, jnp.int32)

        def compute_block(b, ob):
            def group_body(g, _):
                xv = x_v[pl.ds(b * BLK + g * L, L)]
                xs = jnp.clip(xv * jnp.float32(ATOMS), 0.0, float(ATOMS - 1))
                fl = xs.astype(jnp.int32)
                d = xs - fl.astype(jnp.float32)
                bvb[pl.ds(g * L, L)] = fl * D
                dvb[pl.ds(g * L, L)] = d
                return _

            lax.fori_loop(0, BLK // L, group_body, 0, unroll=True)

            @plsc.parallel_loop(0, BLK, unroll=8)
            def elem_body(e):
                es = lax.broadcast(e, (L,))
                bsp = plsc.load_gather(bvb, [es])
                dsp = plsc.load_gather(dvb, [es])
                o0 = e * D
                for c in range(D // L):
                    w = plsc.load_gather(tad_v, [bsp + ka[c]])
                    av = lax.bitcast_convert_type(w & himask, jnp.float32)
                    dv = lax.bitcast_convert_type(
                        lax.shift_left(w, 16), jnp.float32)
                    ob[pl.ds(o0 + c * L, L)] = av + dsp * dv

        def out_slice(b):
            return out_hbm.at[pl.ds((base + b * BLK) * D, BLK * D)]

        def wait(ob, sem):
            pltpu.make_async_copy(ob, out_slice(0), sem).wait()

        def pair_body(i, _):
            b0 = 2 * i

            @pl.when(i > 0)
            def _w0():
                wait(ob0, sem0)

            compute_block(b0, ob0)
            pltpu.async_copy(ob0, out_slice(b0), sem0)

            @pl.when(i > 0)
            def _w1():
                wait(ob1, sem1)

            compute_block(b0 + 1, ob1)
            pltpu.async_copy(ob1, out_slice(b0 + 1), sem1)
            return _

        lax.fori_loop(0, NBLK // 2, pair_body, 0, unroll=False)
        wait(ob0, sem0)
        wait(ob1, sem1)

    return body


_kernel_fn = _make_kernel()


def kernel(x, table):
    xf = x.reshape(N)
    tpad = jnp.concatenate([table, table[-1:]], axis=0)
    a_bits = jax.lax.bitcast_convert_type(
        tpad[:-1].astype(jnp.bfloat16), jnp.uint16).astype(jnp.uint32)
    d_bits = jax.lax.bitcast_convert_type(
        (tpad[1:] - tpad[:-1]).astype(jnp.bfloat16), jnp.uint16
    ).astype(jnp.uint32)
    tad = jax.lax.bitcast_convert_type((a_bits << 16) | d_bits, jnp.int32)
    out = _kernel_fn(xf, tad.reshape(TAD_WORDS))
    return out.reshape(x.shape[0], x.shape[1], D)
